# baseline (speedup 1.0000x reference)
import functools

import jax
import jax.numpy as jnp
from jax import lax
from jax.experimental import pallas as pl
from jax.experimental.pallas import tpu as pltpu
from jax.experimental.pallas import tpu_sc as plsc

N = 50000
F = 128
H = 8
D = 16
OUT = 64
E = 150000

NCHUNK = 8
NPASS = NCHUNK // 2
NB = 128
NBLK = 49
CR = NBLK * NB
NPAD = NCHUNK * CR
EPAD = E + 16
ESL = EPAD // 16
RW = 144


def _proj(x, Wp, bp, Mcat, block=2000):
    bp2 = bp.reshape(1, F)

    def body(x_ref, w_ref, b_ref, m_ref, xr_ref, pk_ref):
        xa = jnp.dot(x_ref[...], w_ref[...],
                     preferred_element_type=jnp.float32) + b_ref[...]
        pk = jnp.dot(xa, m_ref[...], preferred_element_type=jnp.float32)
        xr_ref[:, 0:F] = xa
        xr_ref[:, F:F + 64] = pk
        pk_ref[...] = pk

    return pl.pallas_call(
        body,
        grid=(N // block,),
        in_specs=[
            pl.BlockSpec((block, F), lambda i: (i, 0)),
            pl.BlockSpec((F, F), lambda i: (0, 0)),
            pl.BlockSpec((1, F), lambda i: (0, 0)),
            pl.BlockSpec((F, 64), lambda i: (0, 0)),
        ],
        out_specs=[
            pl.BlockSpec((block, F + 64), lambda i: (i, 0)),
            pl.BlockSpec((block, 64), lambda i: (i, 0)),
        ],
        out_shape=[
            jax.ShapeDtypeStruct((N, F + 64), jnp.float32),
            jax.ShapeDtypeStruct((N, 64), jnp.float32),
        ],
    )(x, Wp, bp2, Mcat)


def _pack_edges(src2d, dst2d):
    def body(s_ref, d_ref, o_ref):
        o_ref[...] = (d_ref[...] << 16) | s_ref[...]

    rows = EPAD // 128
    return pl.pallas_call(
        body,
        out_shape=jax.ShapeDtypeStruct((rows, 128), jnp.int32),
    )(src2d, dst2d)


def _tail_a(acc1, acc2, Wk, bk, EXP8, block=2000):
    bk2 = bk.reshape(1, F)

    def body(a1_ref, a2_ref, wk_ref, bk_ref, e8_ref, o1_ref, o2_ref, ps_ref):
        i = pl.program_id(0)

        @pl.when(i == 0)
        def _():
            ps_ref[...] = jnp.zeros_like(ps_ref)

        e8 = e8_ref[...]
        outs = []
        for a_ref, o_ref in ((a1_ref, o1_ref), (a2_ref, o2_ref)):
            a = a_ref[...]
            recip = 1.0 / (a[:, 128:136] + 1e-16)
            den128 = jnp.dot(recip, e8, preferred_element_type=jnp.float32)
            o = jnp.maximum(a[:, 0:128] * den128, 0.0)
            o_ref[...] = o
            outs.append(o)
        o1, o2 = outs
        t1 = jnp.tanh(jnp.dot(o1, wk_ref[...],
                              preferred_element_type=jnp.float32) + bk_ref[...])
        t2 = jnp.tanh(jnp.dot(o2, wk_ref[...],
                              preferred_element_type=jnp.float32) + bk_ref[...])
        ps = jnp.stack([
            t1.sum(0), t2.sum(0),
            o1.sum(0), o2.sum(0),
            (o1 * o1).sum(0), (o2 * o2).sum(0),
            (o1 * o2).sum(0), jnp.zeros((F,), jnp.float32),
        ])
        ps_ref[...] += ps

    return pl.pallas_call(
        body,
        grid=(N // block,),
        in_specs=[
            pl.BlockSpec((block, RW), lambda i: (i, 0)),
            pl.BlockSpec((block, RW), lambda i: (i, 0)),
            pl.BlockSpec((F, F), lambda i: (0, 0)),
            pl.BlockSpec((1, F), lambda i: (0, 0)),
            pl.BlockSpec((8, F), lambda i: (0, 0)),
        ],
        out_specs=[
            pl.BlockSpec((block, F), lambda i: (i, 0)),
            pl.BlockSpec((block, F), lambda i: (i, 0)),
            pl.BlockSpec((8, F), lambda i: (0, 0)),
        ],
        out_shape=[
            jax.ShapeDtypeStruct((N, F), jnp.float32),
            jax.ShapeDtypeStruct((N, F), jnp.float32),
            jax.ShapeDtypeStruct((8, F), jnp.float32),
        ],
    )(acc1, acc2, Wk, bk2, EXP8)


def _tail_b(o1, o2, W1, W2, b, block=2000):
    b2 = b.reshape(1, OUT)

    def body(o1_ref, o2_ref, w1_ref, w2_ref, b_ref, out_ref):
        out_ref[...] = (
            jnp.dot(o1_ref[...], w1_ref[...], preferred_element_type=jnp.float32)
            + jnp.dot(o2_ref[...], w2_ref[...], preferred_element_type=jnp.float32)
            + b_ref[...]
        )

    return pl.pallas_call(
        body,
        grid=(N // block,),
        in_specs=[
            pl.BlockSpec((block, F), lambda i: (i, 0)),
            pl.BlockSpec((block, F), lambda i: (i, 0)),
            pl.BlockSpec((F, OUT), lambda i: (0, 0)),
            pl.BlockSpec((F, OUT), lambda i: (0, 0)),
            pl.BlockSpec((1, OUT), lambda i: (0, 0)),
        ],
        out_specs=pl.BlockSpec((block, OUT), lambda i: (i, 0)),
        out_shape=jax.ShapeDtypeStruct((N, OUT), jnp.float32),
    )(o1, o2, W1, W2, b2)


def _sc_edge_conv(xra, xrp, pa_t, pp_t, pks):
    mesh = plsc.VectorSubcoreMesh(core_axis_name="c", subcore_axis_name="s")

    @functools.partial(
        pl.kernel, mesh=mesh,
        compiler_params=pltpu.CompilerParams(
            needs_layout_passes=False, use_tc_tiling_on_sc=False),
        out_type=[jax.ShapeDtypeStruct((NPAD, RW), jnp.float32)] * 4,
        scratch_types=[
            pltpu.VMEM((ESL,), jnp.int32),
            pltpu.VMEM((ESL + 16,), jnp.int32),
            pltpu.VMEM((NB, F + 64), jnp.float32),
            pltpu.VMEM((NB, 16), jnp.float32),
            pltpu.VMEM((NB, RW), jnp.float32),
            pltpu.VMEM((32, RW), jnp.float32),
            pltpu.VMEM((NB,), jnp.int32),
            pltpu.VMEM((NB,), jnp.int32),
            pltpu.VMEM((1, NB), jnp.int32),
            pltpu.VMEM_SHARED((CR, RW), jnp.float32),
            pltpu.SemaphoreType.DMA,
            pltpu.SemaphoreType.DMA,
        ],
    )
    def k(xra_h, xrp_h, pa_h, pp_h,
          p_aa, p_ap, p_pa, p_pp,
          acc_aa, acc_ap, acc_pa, acc_pp,
          pksl, edc, rows_v, prd, staging, zbuf,
          gidx, didx, lidx, shared, sem0, sem1):
        core = lax.axis_index("c")
        sub = lax.axis_index("s")
        lane = lax.iota(jnp.int32, 16)
        zi = jnp.zeros((16,), jnp.int32)
        zf = jnp.zeros((16,), jnp.float32)

        def z0(i, _):
            edc[pl.ds(i * 16, 16)] = zi
            return 0
        lax.fori_loop(0, (ESL + 16) // 16, z0, 0)

        def z1(r, _):
            for cblk in range(RW // 16):
                zbuf[r, pl.ds(cblk * 16, 16)] = zf
            return 0
        lax.fori_loop(0, 32, z1, 0)

        cfg = [
            (p_aa, xra_h, 0, pa_h, 1, acc_aa),
            (p_ap, xra_h, 2, pp_h, 0, acc_ap),
            (p_pa, xrp_h, 1, pa_h, 3, acc_pa),
            (p_pp, xrp_h, 2, pp_h, 3, acc_pp),
        ]

        for pk_h, xs_h, gs, dtab, gd, out_h in cfg:
            pltpu.sync_copy(pk_h.at[pl.ds(sub * ESL, ESL)], pksl)

            for cpass in range(NPASS):
                c = core * NPASS + cpass
                lo = c * CR
                hi = lo + CR

                for bi in range((NBLK + 15) // 16):
                    bid = sub + bi * 16

                    @pl.when(bid < NBLK)
                    def _():
                        for qi in range(NB // 32):
                            pltpu.sync_copy(
                                zbuf,
                                shared.at[pl.ds(bid * NB + qi * 32, 32)])
                plsc.subcore_barrier()

                def scan_body(i, cnt):
                    pvec = pksl[pl.ds(i * 16, 16)]
                    dvec = jnp.bitwise_and(pvec >> 16, 0xffff)
                    m = (dvec >= lo) & (dvec < hi)
                    pref = jnp.zeros((16,), jnp.int32)
                    for j in range(1, 16):
                        mj = m & (lane < j)
                        pref = pref + jnp.where(
                            lane == j, plsc.all_reduce_population_count(mj), 0)
                    pos = jnp.where(m, cnt + pref, ESL + lane)
                    plsc.store_scatter(edc, [pos], pvec)
                    return cnt + plsc.all_reduce_population_count(m)[0]
                kk = lax.fori_loop(0, ESL // 16, scan_body, jnp.int32(0))

                def batch_body(j, _):
                    base = j * NB
                    for v in range(NB // 16):
                        off = base + v * 16
                        pv = edc[pl.ds(off, 16)]
                        sv = jnp.bitwise_and(pv, 0xffff)
                        dv = jnp.bitwise_and(pv >> 16, 0xffff)
                        valid = (off + lane) < kk
                        gidx[pl.ds(v * 16, 16)] = sv
                        didx[pl.ds(v * 16, 16)] = dv * 4 + gd
                        lidx[0, pl.ds(v * 16, 16)] = jnp.where(valid, dv - lo, 0)
                    cp0 = pltpu.async_copy(xs_h.at[gidx], rows_v, sem0)
                    cp1 = pltpu.async_copy(dtab.at[didx], prd, sem1)
                    cp0.wait()
                    cp1.wait()

                    def edge_body(e, _):
                        valid = (base + e) < kk
                        al = rows_v[e, pl.ds(F + gs * 16, 16)] + prd[e, :]
                        al = jnp.where(al >= 0, al, 0.2 * al)
                        ex = jnp.exp(al)
                        exm = jnp.where((lane < 8) & valid, ex, 0.0)
                        staging[e, pl.ds(128, 16)] = exm
                        for h in range(8):
                            sh = jnp.broadcast_to(exm[h], (16,))
                            staging[e, pl.ds(h * 16, 16)] = (
                                sh * rows_v[e, pl.ds(h * 16, 16)])
                        return 0
                    lax.fori_loop(0, NB, edge_body, 0)

                    pltpu.sync_copy(staging, shared.at[lidx.at[0]], add=True)
                    return 0
                nb = (kk + NB - 1) // NB
                lax.fori_loop(0, nb, batch_body, 0)
                plsc.subcore_barrier()

                for bi in range((NBLK + 15) // 16):
                    bid = sub + bi * 16

                    @pl.when(bid < NBLK)
                    def _():
                        pltpu.sync_copy(shared.at[pl.ds(bid * NB, NB)], staging)
                        pltpu.sync_copy(
                            staging, out_h.at[pl.ds(lo + bid * NB, NB)])
                plsc.subcore_barrier()

    return k(xra, xrp, pa_t, pp_t,
             pks['aa'], pks['ap'], pks['pa'], pks['pp'])


def _coef_mat(a):
    m = (jnp.eye(H, dtype=jnp.float32)[:, None, :] * a[:, :, None]).reshape(F, H)
    return jnp.pad(m, ((0, 0), (0, 8)))


def kernel(x_author, x_paper, ei_aa, ei_ap, ei_pa, ei_pp,
           Wp_author, bp_author, Wp_paper, bp_paper,
           as_aa, ad_aa, as_ap, ad_ap, as_pa, ad_pa, as_pp, ad_pp,
           Wk, bk, q,
           gnw_author, gnb_author, gns_author, gnw_paper, gnb_paper, gns_paper,
           Wl_author, bl_author, Wl_paper, bl_paper):
    Mcat_a = jnp.concatenate(
        [_coef_mat(as_aa), _coef_mat(ad_aa), _coef_mat(as_ap), _coef_mat(ad_pa)],
        axis=1)
    Mcat_p = jnp.concatenate(
        [_coef_mat(ad_ap), _coef_mat(as_pa), _coef_mat(as_pp), _coef_mat(ad_pp)],
        axis=1)

    xra, pack_a = _proj(x_author, Wp_author, bp_author, Mcat_a)
    xrp, pack_p = _proj(x_paper, Wp_paper, bp_paper, Mcat_p)
    pa_t = pack_a.reshape(N * 4, 16)
    pp_t = pack_p.reshape(N * 4, 16)

    pks = {}
    for name, ei in (('aa', ei_aa), ('ap', ei_ap), ('pa', ei_pa), ('pp', ei_pp)):
        src = jnp.pad(ei[0].astype(jnp.int32), (0, EPAD - E))
        dst = jnp.pad(ei[1].astype(jnp.int32), (0, EPAD - E),
                      constant_values=0xffff)
        pks[name] = _pack_edges(
            src.reshape(EPAD // 128, 128),
            dst.reshape(EPAD // 128, 128)).reshape(EPAD)

    acc_aa, acc_ap, acc_pa, acc_pp = (
        a[:N] for a in _sc_edge_conv(xra, xrp, pa_t, pp_t, pks))

    EXP8 = jnp.repeat(jnp.eye(8, dtype=jnp.float32), 16, axis=1)

    outs = {}
    for nt, a1, a2, gw, gb, gms, Wl, bl in (
            ('author', acc_aa, acc_pa, gnw_author, gnb_author, gns_author,
             Wl_author, bl_author),
            ('paper', acc_ap, acc_pp, gnw_paper, gnb_paper, gns_paper,
             Wl_paper, bl_paper)):
        o1, o2, ps = _tail_a(a1, a2, Wk, bk, EXP8)
        t1, t2, S1, S2, Q1, Q2, X12 = (ps[0], ps[1], ps[2], ps[3], ps[4],
                                       ps[5], ps[6])
        k1 = t1 / N
        k2 = t2 / N
        score = jnp.stack([(q * k1).sum(), (q * k2).sum()])
        attn = jax.nn.softmax(score)
        a0, a1s = attn[0], attn[1]
        mean = (a0 * S1 + a1s * S2) / N
        Eo2 = (a0 * a0 * Q1 + a1s * a1s * Q2 + 2 * a0 * a1s * X12) / N
        var = Eo2 - 2 * gms * mean * mean + gms * gms * mean * mean
        scale = gw / jnp.sqrt(var + 1e-5)
        shift = gb - scale * mean * gms
        Wl1 = scale[:, None] * Wl
        bl1 = shift @ Wl + bl
        outs[nt] = _tail_b(o1, o2, a0 * Wl1, a1s * Wl1, bl1)

    return (outs['author'], outs['paper'])


# speedup vs baseline: 1.5646x; 1.5646x over previous
import functools

import jax
import jax.numpy as jnp
from jax import lax
from jax.experimental import pallas as pl
from jax.experimental.pallas import tpu as pltpu
from jax.experimental.pallas import tpu_sc as plsc

N = 50000
F = 128
H = 8
D = 16
OUT = 64
E = 150000

NCHUNK = 8
NPASS = NCHUNK // 2
NB = 128
NBLK = 49
CR = NBLK * NB
NPAD = NCHUNK * CR
EPAD = E + 16
ESL = EPAD // 16
EDC = ((ESL + NB - 1) // NB) * NB
RW = 144


def _proj(x, Wp, bp, Mcat, block=2000):
    bp2 = bp.reshape(1, F)

    def body(x_ref, w_ref, b_ref, m_ref, xa_ref, pk_ref):
        xa = jnp.dot(x_ref[...], w_ref[...],
                     preferred_element_type=jnp.float32) + b_ref[...]
        xa_ref[...] = xa
        pk_ref[...] = jnp.dot(xa, m_ref[...], preferred_element_type=jnp.float32)

    return pl.pallas_call(
        body,
        grid=(N // block,),
        in_specs=[
            pl.BlockSpec((block, F), lambda i: (i, 0)),
            pl.BlockSpec((F, F), lambda i: (0, 0)),
            pl.BlockSpec((1, F), lambda i: (0, 0)),
            pl.BlockSpec((F, 64), lambda i: (0, 0)),
        ],
        out_specs=[
            pl.BlockSpec((block, F), lambda i: (i, 0)),
            pl.BlockSpec((block, 64), lambda i: (i, 0)),
        ],
        out_shape=[
            jax.ShapeDtypeStruct((N, F), jnp.float32),
            jax.ShapeDtypeStruct((N, 64), jnp.float32),
        ],
    )(x, Wp, bp2, Mcat)


def _pack_edges(src2d, dst2d):
    def body(s_ref, d_ref, o_ref):
        o_ref[...] = (d_ref[...] << 16) | s_ref[...]

    rows = EPAD // 128
    return pl.pallas_call(
        body,
        out_shape=jax.ShapeDtypeStruct((rows, 128), jnp.int32),
    )(src2d, dst2d)


def _tail_a(acc1, acc2, Wk, bk, EXP8, block=2000):
    bk2 = bk.reshape(1, F)

    def body(a1_ref, a2_ref, wk_ref, bk_ref, e8_ref, o1_ref, o2_ref, ps_ref):
        i = pl.program_id(0)

        @pl.when(i == 0)
        def _():
            ps_ref[...] = jnp.zeros_like(ps_ref)

        e8 = e8_ref[...]
        outs = []
        for a_ref, o_ref in ((a1_ref, o1_ref), (a2_ref, o2_ref)):
            a = a_ref[...]
            recip = 1.0 / (a[:, 128:136] + 1e-16)
            den128 = jnp.dot(recip, e8, preferred_element_type=jnp.float32)
            o = jnp.maximum(a[:, 0:128] * den128, 0.0)
            o_ref[...] = o
            outs.append(o)
        o1, o2 = outs
        t1 = jnp.tanh(jnp.dot(o1, wk_ref[...],
                              preferred_element_type=jnp.float32) + bk_ref[...])
        t2 = jnp.tanh(jnp.dot(o2, wk_ref[...],
                              preferred_element_type=jnp.float32) + bk_ref[...])
        ps = jnp.stack([
            t1.sum(0), t2.sum(0),
            o1.sum(0), o2.sum(0),
            (o1 * o1).sum(0), (o2 * o2).sum(0),
            (o1 * o2).sum(0), jnp.zeros((F,), jnp.float32),
        ])
        ps_ref[...] += ps

    return pl.pallas_call(
        body,
        grid=(N // block,),
        in_specs=[
            pl.BlockSpec((block, RW), lambda i: (i, 0)),
            pl.BlockSpec((block, RW), lambda i: (i, 0)),
            pl.BlockSpec((F, F), lambda i: (0, 0)),
            pl.BlockSpec((1, F), lambda i: (0, 0)),
            pl.BlockSpec((8, F), lambda i: (0, 0)),
        ],
        out_specs=[
            pl.BlockSpec((block, F), lambda i: (i, 0)),
            pl.BlockSpec((block, F), lambda i: (i, 0)),
            pl.BlockSpec((8, F), lambda i: (0, 0)),
        ],
        out_shape=[
            jax.ShapeDtypeStruct((N, F), jnp.float32),
            jax.ShapeDtypeStruct((N, F), jnp.float32),
            jax.ShapeDtypeStruct((8, F), jnp.float32),
        ],
    )(acc1, acc2, Wk, bk2, EXP8)


def _tail_b(o1, o2, W1, W2, b, block=2000):
    b2 = b.reshape(1, OUT)

    def body(o1_ref, o2_ref, w1_ref, w2_ref, b_ref, out_ref):
        out_ref[...] = (
            jnp.dot(o1_ref[...], w1_ref[...], preferred_element_type=jnp.float32)
            + jnp.dot(o2_ref[...], w2_ref[...], preferred_element_type=jnp.float32)
            + b_ref[...]
        )

    return pl.pallas_call(
        body,
        grid=(N // block,),
        in_specs=[
            pl.BlockSpec((block, F), lambda i: (i, 0)),
            pl.BlockSpec((block, F), lambda i: (i, 0)),
            pl.BlockSpec((F, OUT), lambda i: (0, 0)),
            pl.BlockSpec((F, OUT), lambda i: (0, 0)),
            pl.BlockSpec((1, OUT), lambda i: (0, 0)),
        ],
        out_specs=pl.BlockSpec((block, OUT), lambda i: (i, 0)),
        out_shape=jax.ShapeDtypeStruct((N, OUT), jnp.float32),
    )(o1, o2, W1, W2, b2)


def _sc_edge_conv(xa, xp, pa_t, pp_t, pks):
    mesh = plsc.VectorSubcoreMesh(core_axis_name="c", subcore_axis_name="s")

    @functools.partial(
        pl.kernel, mesh=mesh,
        compiler_params=pltpu.CompilerParams(
            needs_layout_passes=False, use_tc_tiling_on_sc=False),
        out_type=[jax.ShapeDtypeStruct((NPAD, RW), jnp.float32)] * 4,
        scratch_types=[
            pltpu.VMEM((ESL,), jnp.int32),
            pltpu.VMEM((EDC + 16,), jnp.int32),
            pltpu.VMEM((NB, F), jnp.float32),
            pltpu.VMEM((NB, 16), jnp.float32),
            pltpu.VMEM((NB, 16), jnp.float32),
            pltpu.VMEM((NB, RW), jnp.float32),
            pltpu.VMEM((32, RW), jnp.float32),
            pltpu.VMEM((NB,), jnp.int32),
            pltpu.VMEM((NB,), jnp.int32),
            pltpu.VMEM((NB,), jnp.int32),
            pltpu.VMEM((1, NB), jnp.int32),
            pltpu.VMEM_SHARED((CR, RW), jnp.float32),
            pltpu.SemaphoreType.DMA,
            pltpu.SemaphoreType.DMA,
            pltpu.SemaphoreType.DMA,
        ],
    )
    def k(xa_h, xp_h, pa_h, pp_h,
          p_aa, p_ap, p_pa, p_pp,
          acc_aa, acc_ap, acc_pa, acc_pp,
          pksl, edc, rows_v, prs, prd, staging, zbuf,
          gidx, sidx, didx, lidx, shared, sem0, sem1, sem2):
        core = lax.axis_index("c")
        sub = lax.axis_index("s")
        lane = lax.iota(jnp.int32, 16)
        zi = jnp.zeros((16,), jnp.int32)
        zf = jnp.zeros((16,), jnp.float32)

        def z0(i, _):
            edc[pl.ds(i * 16, 16)] = zi
            return 0
        lax.fori_loop(0, (EDC + 16) // 16, z0, 0)

        def z1(r, _):
            for cblk in range(RW // 16):
                zbuf[r, pl.ds(cblk * 16, 16)] = zf
            return 0
        lax.fori_loop(0, 32, z1, 0)

        cfg = [
            (p_aa, xa_h, pa_h, 0, pa_h, 1, acc_aa),
            (p_ap, xa_h, pa_h, 2, pp_h, 0, acc_ap),
            (p_pa, xp_h, pp_h, 1, pa_h, 3, acc_pa),
            (p_pp, xp_h, pp_h, 2, pp_h, 3, acc_pp),
        ]

        for pk_h, xs_h, stab, gs, dtab, gd, out_h in cfg:
            pltpu.sync_copy(pk_h.at[pl.ds(sub * ESL, ESL)], pksl)

            def cpass_body(cpass, _unused):
                c = core * NPASS + cpass
                lo = c * CR
                hi = lo + CR

                for bi in range((NBLK + 15) // 16):
                    bid = sub + bi * 16

                    @pl.when(bid < NBLK)
                    def _():
                        for qi in range(NB // 32):
                            pltpu.sync_copy(
                                zbuf,
                                shared.at[pl.ds(bid * NB + qi * 32, 32)])
                plsc.subcore_barrier()

                @plsc.parallel_loop(0, ESL // 16, unroll=4,
                                    carry=jnp.int32(0))
                def scan_body(i, cnt):
                    pvec = pksl[pl.ds(i * 16, 16)]
                    dvec = jnp.bitwise_and(pvec >> 16, 0xffff)
                    m = (dvec >= lo) & (dvec < hi)
                    pref = jnp.zeros((16,), jnp.int32)
                    for j in range(1, 16):
                        mj = m & (lane < j)
                        pref = pref + jnp.where(
                            lane == j, plsc.all_reduce_population_count(mj), 0)
                    pos = jnp.where(m, cnt + pref, EDC + lane)
                    plsc.store_scatter(edc, [pos], pvec)
                    return cnt + plsc.all_reduce_population_count(m)[0]
                kk = scan_body

                def batch_body(j, _):
                    base = j * NB
                    for v in range(NB // 16):
                        off = base + v * 16
                        pv = edc[pl.ds(off, 16)]
                        sv = jnp.bitwise_and(pv, 0xffff)
                        dv = jnp.bitwise_and(pv >> 16, 0xffff)
                        valid = (off + lane) < kk
                        gidx[pl.ds(v * 16, 16)] = sv
                        sidx[pl.ds(v * 16, 16)] = sv * 4 + gs
                        didx[pl.ds(v * 16, 16)] = dv * 4 + gd
                        lidx[0, pl.ds(v * 16, 16)] = jnp.where(valid, dv - lo, 0)
                    cp0 = pltpu.async_copy(xs_h.at[gidx], rows_v, sem0)
                    cp1 = pltpu.async_copy(stab.at[sidx], prs, sem1)
                    cp2 = pltpu.async_copy(dtab.at[didx], prd, sem2)
                    cp0.wait()
                    cp1.wait()
                    cp2.wait()

                    @plsc.parallel_loop(0, NB, unroll=2)
                    def edge_body(e):
                        valid = (base + e) < kk
                        al = prs[e, :] + prd[e, :]
                        al = jnp.where(al >= 0, al, 0.2 * al)
                        ex = jnp.exp(al)
                        exm = jnp.where((lane < 8) & valid, ex, 0.0)
                        staging[e, pl.ds(128, 16)] = exm
                        for h in range(8):
                            sh = jnp.broadcast_to(exm[h], (16,))
                            staging[e, pl.ds(h * 16, 16)] = (
                                sh * rows_v[e, pl.ds(h * 16, 16)])

                    pltpu.sync_copy(staging, shared.at[lidx.at[0]], add=True)
                    return 0
                nb = (kk + NB - 1) // NB
                lax.fori_loop(0, nb, batch_body, 0)
                plsc.subcore_barrier()

                for bi in range((NBLK + 15) // 16):
                    bid = sub + bi * 16

                    @pl.when(bid < NBLK)
                    def _():
                        pltpu.sync_copy(shared.at[pl.ds(bid * NB, NB)], staging)
                        pltpu.sync_copy(
                            staging, out_h.at[pl.ds(lo + bid * NB, NB)])
                plsc.subcore_barrier()
                return 0

            lax.fori_loop(0, NPASS, cpass_body, 0)

    return k(xa, xp, pa_t, pp_t,
             pks['aa'], pks['ap'], pks['pa'], pks['pp'])


def _coef_mat(a):
    m = (jnp.eye(H, dtype=jnp.float32)[:, None, :] * a[:, :, None]).reshape(F, H)
    return jnp.pad(m, ((0, 0), (0, 8)))


def kernel(x_author, x_paper, ei_aa, ei_ap, ei_pa, ei_pp,
           Wp_author, bp_author, Wp_paper, bp_paper,
           as_aa, ad_aa, as_ap, ad_ap, as_pa, ad_pa, as_pp, ad_pp,
           Wk, bk, q,
           gnw_author, gnb_author, gns_author, gnw_paper, gnb_paper, gns_paper,
           Wl_author, bl_author, Wl_paper, bl_paper):
    Mcat_a = jnp.concatenate(
        [_coef_mat(as_aa), _coef_mat(ad_aa), _coef_mat(as_ap), _coef_mat(ad_pa)],
        axis=1)
    Mcat_p = jnp.concatenate(
        [_coef_mat(ad_ap), _coef_mat(as_pa), _coef_mat(as_pp), _coef_mat(ad_pp)],
        axis=1)

    xa, pack_a = _proj(x_author, Wp_author, bp_author, Mcat_a)
    xp, pack_p = _proj(x_paper, Wp_paper, bp_paper, Mcat_p)
    pa_t = pack_a.reshape(N * 4, 16)
    pp_t = pack_p.reshape(N * 4, 16)

    pks = {}
    for name, ei in (('aa', ei_aa), ('ap', ei_ap), ('pa', ei_pa), ('pp', ei_pp)):
        src = jnp.pad(ei[0].astype(jnp.int32), (0, EPAD - E))
        dst = jnp.pad(ei[1].astype(jnp.int32), (0, EPAD - E),
                      constant_values=0xffff)
        pks[name] = _pack_edges(
            src.reshape(EPAD // 128, 128),
            dst.reshape(EPAD // 128, 128)).reshape(EPAD)

    acc_aa, acc_ap, acc_pa, acc_pp = (
        a[:N] for a in _sc_edge_conv(xa, xp, pa_t, pp_t, pks))

    EXP8 = jnp.repeat(jnp.eye(8, dtype=jnp.float32), 16, axis=1)

    outs = {}
    for nt, a1, a2, gw, gb, gms, Wl, bl in (
            ('author', acc_aa, acc_pa, gnw_author, gnb_author, gns_author,
             Wl_author, bl_author),
            ('paper', acc_ap, acc_pp, gnw_paper, gnb_paper, gns_paper,
             Wl_paper, bl_paper)):
        o1, o2, ps = _tail_a(a1, a2, Wk, bk, EXP8)
        t1, t2, S1, S2, Q1, Q2, X12 = (ps[0], ps[1], ps[2], ps[3], ps[4],
                                       ps[5], ps[6])
        k1 = t1 / N
        k2 = t2 / N
        score = jnp.stack([(q * k1).sum(), (q * k2).sum()])
        attn = jax.nn.softmax(score)
        a0, a1s = attn[0], attn[1]
        mean = (a0 * S1 + a1s * S2) / N
        Eo2 = (a0 * a0 * Q1 + a1s * a1s * Q2 + 2 * a0 * a1s * X12) / N
        var = Eo2 - 2 * gms * mean * mean + gms * gms * mean * mean
        scale = gw / jnp.sqrt(var + 1e-5)
        shift = gb - scale * mean * gms
        Wl1 = scale[:, None] * Wl
        bl1 = shift @ Wl + bl
        outs[nt] = _tail_b(o1, o2, a0 * Wl1, a1s * Wl1, bl1)

    return (outs['author'], outs['paper'])


# speedup vs baseline: 1.5649x; 1.0002x over previous
import functools

import jax
import jax.numpy as jnp
from jax import lax
from jax.experimental import pallas as pl
from jax.experimental.pallas import tpu as pltpu
from jax.experimental.pallas import tpu_sc as plsc

N = 50000
F = 128
H = 8
D = 16
OUT = 64
E = 150000

NCHUNK = 8
NPASS = NCHUNK // 2
NB = 128
NBLK = 49
CR = NBLK * NB
NPAD = NCHUNK * CR
EPAD = E + 16
ESL = EPAD // 16
EDC = ((ESL + NB - 1) // NB) * NB
RW = 144


def _proj(x, Wp, bp, Mcat, block=2000):
    bp2 = bp.reshape(1, F)

    def body(x_ref, w_ref, b_ref, m_ref, xa_ref, pk_ref):
        xa = jnp.dot(x_ref[...], w_ref[...],
                     preferred_element_type=jnp.float32) + b_ref[...]
        xa_ref[...] = xa
        pk_ref[...] = jnp.dot(xa, m_ref[...], preferred_element_type=jnp.float32)

    return pl.pallas_call(
        body,
        grid=(N // block,),
        in_specs=[
            pl.BlockSpec((block, F), lambda i: (i, 0)),
            pl.BlockSpec((F, F), lambda i: (0, 0)),
            pl.BlockSpec((1, F), lambda i: (0, 0)),
            pl.BlockSpec((F, 64), lambda i: (0, 0)),
        ],
        out_specs=[
            pl.BlockSpec((block, F), lambda i: (i, 0)),
            pl.BlockSpec((block, 64), lambda i: (i, 0)),
        ],
        out_shape=[
            jax.ShapeDtypeStruct((N, F), jnp.float32),
            jax.ShapeDtypeStruct((N, 64), jnp.float32),
        ],
    )(x, Wp, bp2, Mcat)


def _pack_edges(src2d, dst2d):
    def body(s_ref, d_ref, o_ref):
        o_ref[...] = (d_ref[...] << 16) | s_ref[...]

    rows = EPAD // 128
    return pl.pallas_call(
        body,
        out_shape=jax.ShapeDtypeStruct((rows, 128), jnp.int32),
    )(src2d, dst2d)


def _tail_a(acc1, acc2, Wk, bk, EXP8, block=2000):
    bk2 = bk.reshape(1, F)

    def body(a1_ref, a2_ref, wk_ref, bk_ref, e8_ref, o1_ref, o2_ref, ps_ref):
        i = pl.program_id(0)

        @pl.when(i == 0)
        def _():
            ps_ref[...] = jnp.zeros_like(ps_ref)

        e8 = e8_ref[...]
        outs = []
        for a_ref, o_ref in ((a1_ref, o1_ref), (a2_ref, o2_ref)):
            a = a_ref[...]
            recip = 1.0 / (a[:, 128:136] + 1e-16)
            den128 = jnp.dot(recip, e8, preferred_element_type=jnp.float32)
            o = jnp.maximum(a[:, 0:128] * den128, 0.0)
            o_ref[...] = o
            outs.append(o)
        o1, o2 = outs
        t1 = jnp.tanh(jnp.dot(o1, wk_ref[...],
                              preferred_element_type=jnp.float32) + bk_ref[...])
        t2 = jnp.tanh(jnp.dot(o2, wk_ref[...],
                              preferred_element_type=jnp.float32) + bk_ref[...])
        ps = jnp.stack([
            t1.sum(0), t2.sum(0),
            o1.sum(0), o2.sum(0),
            (o1 * o1).sum(0), (o2 * o2).sum(0),
            (o1 * o2).sum(0), jnp.zeros((F,), jnp.float32),
        ])
        ps_ref[...] += ps

    return pl.pallas_call(
        body,
        grid=(N // block,),
        in_specs=[
            pl.BlockSpec((block, RW), lambda i: (i, 0)),
            pl.BlockSpec((block, RW), lambda i: (i, 0)),
            pl.BlockSpec((F, F), lambda i: (0, 0)),
            pl.BlockSpec((1, F), lambda i: (0, 0)),
            pl.BlockSpec((8, F), lambda i: (0, 0)),
        ],
        out_specs=[
            pl.BlockSpec((block, F), lambda i: (i, 0)),
            pl.BlockSpec((block, F), lambda i: (i, 0)),
            pl.BlockSpec((8, F), lambda i: (0, 0)),
        ],
        out_shape=[
            jax.ShapeDtypeStruct((N, F), jnp.float32),
            jax.ShapeDtypeStruct((N, F), jnp.float32),
            jax.ShapeDtypeStruct((8, F), jnp.float32),
        ],
    )(acc1, acc2, Wk, bk2, EXP8)


def _tail_b(o1, o2, W1, W2, b, block=2000):
    b2 = b.reshape(1, OUT)

    def body(o1_ref, o2_ref, w1_ref, w2_ref, b_ref, out_ref):
        out_ref[...] = (
            jnp.dot(o1_ref[...], w1_ref[...], preferred_element_type=jnp.float32)
            + jnp.dot(o2_ref[...], w2_ref[...], preferred_element_type=jnp.float32)
            + b_ref[...]
        )

    return pl.pallas_call(
        body,
        grid=(N // block,),
        in_specs=[
            pl.BlockSpec((block, F), lambda i: (i, 0)),
            pl.BlockSpec((block, F), lambda i: (i, 0)),
            pl.BlockSpec((F, OUT), lambda i: (0, 0)),
            pl.BlockSpec((F, OUT), lambda i: (0, 0)),
            pl.BlockSpec((1, OUT), lambda i: (0, 0)),
        ],
        out_specs=pl.BlockSpec((block, OUT), lambda i: (i, 0)),
        out_shape=jax.ShapeDtypeStruct((N, OUT), jnp.float32),
    )(o1, o2, W1, W2, b2)


def _sc_edge_conv(xa, xp, pa_t, pp_t, pks):
    mesh = plsc.VectorSubcoreMesh(core_axis_name="c", subcore_axis_name="s")

    @functools.partial(
        pl.kernel, mesh=mesh,
        compiler_params=pltpu.CompilerParams(
            needs_layout_passes=False, use_tc_tiling_on_sc=False),
        out_type=[jax.ShapeDtypeStruct((NPAD, RW), jnp.float32)] * 4,
        scratch_types=[
            pltpu.VMEM((ESL,), jnp.int32),
            pltpu.VMEM((EDC + 16,), jnp.int32),
            pltpu.VMEM((NB, F), jnp.float32),
            pltpu.VMEM((NB, 16), jnp.float32),
            pltpu.VMEM((NB, 16), jnp.float32),
            pltpu.VMEM((NB, RW), jnp.float32),
            pltpu.VMEM((32, RW), jnp.float32),
            pltpu.VMEM((NB,), jnp.int32),
            pltpu.VMEM((NB,), jnp.int32),
            pltpu.VMEM((NB,), jnp.int32),
            pltpu.VMEM((1, NB), jnp.int32),
            pltpu.VMEM_SHARED((CR, RW), jnp.float32),
            pltpu.SemaphoreType.DMA,
            pltpu.SemaphoreType.DMA,
            pltpu.SemaphoreType.DMA,
        ],
    )
    def k(xa_h, xp_h, pa_h, pp_h,
          p_aa, p_ap, p_pa, p_pp,
          acc_aa, acc_ap, acc_pa, acc_pp,
          pksl, edc, rows_v, prs, prd, staging, zbuf,
          gidx, sidx, didx, lidx, shared, sem0, sem1, sem2):
        core = lax.axis_index("c")
        sub = lax.axis_index("s")
        lane = lax.iota(jnp.int32, 16)
        zi = jnp.zeros((16,), jnp.int32)
        zf = jnp.zeros((16,), jnp.float32)

        def z0(i, _):
            edc[pl.ds(i * 16, 16)] = zi
            return 0
        lax.fori_loop(0, (EDC + 16) // 16, z0, 0)

        def z1(r, _):
            for cblk in range(RW // 16):
                zbuf[r, pl.ds(cblk * 16, 16)] = zf
            return 0
        lax.fori_loop(0, 32, z1, 0)

        cfg = [
            (p_aa, xa_h, pa_h, 0, pa_h, 1, acc_aa),
            (p_ap, xa_h, pa_h, 2, pp_h, 0, acc_ap),
            (p_pa, xp_h, pp_h, 1, pa_h, 3, acc_pa),
            (p_pp, xp_h, pp_h, 2, pp_h, 3, acc_pp),
        ]

        for pk_h, xs_h, stab, gs, dtab, gd, out_h in cfg:
            pltpu.sync_copy(pk_h.at[pl.ds(sub * ESL, ESL)], pksl)

            def cpass_body(cpass, _unused):
                c = core * NPASS + cpass
                lo = c * CR
                hi = lo + CR

                for bi in range((NBLK + 15) // 16):
                    bid = sub + bi * 16

                    @pl.when(bid < NBLK)
                    def _():
                        for qi in range(NB // 32):
                            pltpu.sync_copy(
                                zbuf,
                                shared.at[pl.ds(bid * NB + qi * 32, 32)])
                plsc.subcore_barrier()

                @plsc.parallel_loop(0, ESL // 16, unroll=4,
                                    carry=jnp.int32(0))
                def scan_body(i, cnt):
                    pvec = pksl[pl.ds(i * 16, 16)]
                    dvec = jnp.bitwise_and(pvec >> 16, 0xffff)
                    m = (dvec >= lo) & (dvec < hi)
                    pref = jnp.zeros((16,), jnp.int32)
                    for j in range(1, 16):
                        mj = m & (lane < j)
                        pref = pref + jnp.where(
                            lane == j, plsc.all_reduce_population_count(mj), 0)
                    pos = jnp.where(m, cnt + pref, EDC + lane)
                    plsc.store_scatter(edc, [pos], pvec)
                    return cnt + plsc.all_reduce_population_count(m)[0]
                kk = scan_body

                def batch_body(j, _):
                    base = j * NB
                    for v in range(NB // 16):
                        off = base + v * 16
                        pv = edc[pl.ds(off, 16)]
                        sv = jnp.bitwise_and(pv, 0xffff)
                        dv = jnp.bitwise_and(pv >> 16, 0xffff)
                        valid = (off + lane) < kk
                        gidx[pl.ds(v * 16, 16)] = sv
                        sidx[pl.ds(v * 16, 16)] = sv * 4 + gs
                        didx[pl.ds(v * 16, 16)] = dv * 4 + gd
                        lidx[0, pl.ds(v * 16, 16)] = jnp.where(valid, dv - lo, 0)
                    cp0 = pltpu.async_copy(xs_h.at[gidx], rows_v, sem0)
                    cp1 = pltpu.async_copy(stab.at[sidx], prs, sem1)
                    cp2 = pltpu.async_copy(dtab.at[didx], prd, sem2)
                    cp0.wait()
                    cp1.wait()
                    cp2.wait()

                    @plsc.parallel_loop(0, NB, unroll=4)
                    def edge_body(e):
                        valid = (base + e) < kk
                        al = prs[e, :] + prd[e, :]
                        al = jnp.where(al >= 0, al, 0.2 * al)
                        ex = jnp.exp(al)
                        exm = jnp.where((lane < 8) & valid, ex, 0.0)
                        staging[e, pl.ds(128, 16)] = exm
                        for h in range(8):
                            sh = jnp.broadcast_to(exm[h], (16,))
                            staging[e, pl.ds(h * 16, 16)] = (
                                sh * rows_v[e, pl.ds(h * 16, 16)])

                    pltpu.sync_copy(staging, shared.at[lidx.at[0]], add=True)
                    return 0
                nb = (kk + NB - 1) // NB
                lax.fori_loop(0, nb, batch_body, 0)
                plsc.subcore_barrier()

                for bi in range((NBLK + 15) // 16):
                    bid = sub + bi * 16

                    @pl.when(bid < NBLK)
                    def _():
                        pltpu.sync_copy(shared.at[pl.ds(bid * NB, NB)], staging)
                        pltpu.sync_copy(
                            staging, out_h.at[pl.ds(lo + bid * NB, NB)])
                plsc.subcore_barrier()
                return 0

            lax.fori_loop(0, NPASS, cpass_body, 0)

    return k(xa, xp, pa_t, pp_t,
             pks['aa'], pks['ap'], pks['pa'], pks['pp'])


def _coef_mat(a):
    m = (jnp.eye(H, dtype=jnp.float32)[:, None, :] * a[:, :, None]).reshape(F, H)
    return jnp.pad(m, ((0, 0), (0, 8)))


def kernel(x_author, x_paper, ei_aa, ei_ap, ei_pa, ei_pp,
           Wp_author, bp_author, Wp_paper, bp_paper,
           as_aa, ad_aa, as_ap, ad_ap, as_pa, ad_pa, as_pp, ad_pp,
           Wk, bk, q,
           gnw_author, gnb_author, gns_author, gnw_paper, gnb_paper, gns_paper,
           Wl_author, bl_author, Wl_paper, bl_paper):
    Mcat_a = jnp.concatenate(
        [_coef_mat(as_aa), _coef_mat(ad_aa), _coef_mat(as_ap), _coef_mat(ad_pa)],
        axis=1)
    Mcat_p = jnp.concatenate(
        [_coef_mat(ad_ap), _coef_mat(as_pa), _coef_mat(as_pp), _coef_mat(ad_pp)],
        axis=1)

    xa, pack_a = _proj(x_author, Wp_author, bp_author, Mcat_a)
    xp, pack_p = _proj(x_paper, Wp_paper, bp_paper, Mcat_p)
    pa_t = pack_a.reshape(N * 4, 16)
    pp_t = pack_p.reshape(N * 4, 16)

    pks = {}
    for name, ei in (('aa', ei_aa), ('ap', ei_ap), ('pa', ei_pa), ('pp', ei_pp)):
        src = jnp.pad(ei[0].astype(jnp.int32), (0, EPAD - E))
        dst = jnp.pad(ei[1].astype(jnp.int32), (0, EPAD - E),
                      constant_values=0xffff)
        pks[name] = _pack_edges(
            src.reshape(EPAD // 128, 128),
            dst.reshape(EPAD // 128, 128)).reshape(EPAD)

    acc_aa, acc_ap, acc_pa, acc_pp = (
        a[:N] for a in _sc_edge_conv(xa, xp, pa_t, pp_t, pks))

    EXP8 = jnp.repeat(jnp.eye(8, dtype=jnp.float32), 16, axis=1)

    outs = {}
    for nt, a1, a2, gw, gb, gms, Wl, bl in (
            ('author', acc_aa, acc_pa, gnw_author, gnb_author, gns_author,
             Wl_author, bl_author),
            ('paper', acc_ap, acc_pp, gnw_paper, gnb_paper, gns_paper,
             Wl_paper, bl_paper)):
        o1, o2, ps = _tail_a(a1, a2, Wk, bk, EXP8)
        t1, t2, S1, S2, Q1, Q2, X12 = (ps[0], ps[1], ps[2], ps[3], ps[4],
                                       ps[5], ps[6])
        k1 = t1 / N
        k2 = t2 / N
        score = jnp.stack([(q * k1).sum(), (q * k2).sum()])
        attn = jax.nn.softmax(score)
        a0, a1s = attn[0], attn[1]
        mean = (a0 * S1 + a1s * S2) / N
        Eo2 = (a0 * a0 * Q1 + a1s * a1s * Q2 + 2 * a0 * a1s * X12) / N
        var = Eo2 - 2 * gms * mean * mean + gms * gms * mean * mean
        scale = gw / jnp.sqrt(var + 1e-5)
        shift = gb - scale * mean * gms
        Wl1 = scale[:, None] * Wl
        bl1 = shift @ Wl + bl
        outs[nt] = _tail_b(o1, o2, a0 * Wl1, a1s * Wl1, bl1)

    return (outs['author'], outs['paper'])
